# baseline (speedup 1.0000x reference)
import functools

import jax
import jax.numpy as jnp
from jax import lax
from jax.experimental import pallas as pl
from jax.experimental.pallas import tpu as pltpu
from jax.experimental.pallas import tpu_sc as plsc

_NC = 2
_NS = 16
_NW = _NC * _NS
_MAGIC = 0x5F3759DF


def _cost16(x, y):
    qx = jnp.minimum(jnp.maximum(jnp.abs(x) * 4.0 - 3.0, 0.0), 30.0)
    qy = jnp.minimum(jnp.maximum(jnp.abs(y) * 4.0 - 3.0, 0.0), 30.0)
    s = x * x + y * y
    si = lax.bitcast_convert_type(s, jnp.int32)
    r = lax.bitcast_convert_type(_MAGIC - (si >> 1), jnp.float32)
    h = 0.5 * s
    r = r * (1.5 - h * r * r)
    r = r * (1.5 - h * r * r)
    rt = s * r
    j = jnp.minimum(jnp.maximum(rt * 8.0 - 1.0, 0.0), 20.0)
    return 0.1 * (qx * qx + qy * qy + j * j)


_mesh = plsc.VectorSubcoreMesh(core_axis_name="c", subcore_axis_name="s")


@functools.partial(
    pl.kernel,
    mesh=_mesh,
    out_type=jax.ShapeDtypeStruct((8, 128, 8, 128), jnp.float32),
    scratch_types=[
        pltpu.VMEM((8, 8, 2, 128), jnp.float32),
        pltpu.VMEM((8, 8, 128), jnp.float32),
        pltpu.SemaphoreType.DMA,
    ],
)
def _sc_cost(x4, o4, in_buf, out_buf, sem):
    wid = lax.axis_index("s") * _NC + lax.axis_index("c")
    rtile = wid // 4
    quarter = wid % 4
    for g in range(4):
        ct = quarter * 32 + g * 8
        pltpu.async_copy(
            x4.at[pl.ds(rtile * 8, 8), pl.ds(ct, 8)], in_buf, sem
        ).wait()

        def body(k, _):
            i = k // 8
            j = k % 8
            for l in range(8):
                x = in_buf[i, j, 0, pl.ds(l * 16, 16)]
                y = in_buf[i, j, 1, pl.ds(l * 16, 16)]
                out_buf[j, i, pl.ds(l * 16, 16)] = _cost16(x, y)
            return _

        lax.fori_loop(0, 64, body, None)
        pltpu.async_copy(out_buf, o4.at[rtile, pl.ds(ct, 8)], sem).wait()


def kernel(trajs):
    b, n, _ = trajs.shape
    x4 = trajs.reshape(b, n // 128, 128, 2).transpose(0, 1, 3, 2)
    o4 = _sc_cost(x4)
    return o4.transpose(0, 2, 1, 3).reshape(b, n)


# speedup vs baseline: 1.8647x; 1.8647x over previous
import functools

import jax
import jax.numpy as jnp
from jax import lax
from jax.experimental import pallas as pl
from jax.experimental.pallas import tpu as pltpu
from jax.experimental.pallas import tpu_sc as plsc

_NC = 2
_NS = 16
_MAGIC = 0x5F3759DF


def _cost16(x, y):
    qx = jnp.minimum(jnp.maximum(jnp.abs(x) * 4.0 - 3.0, 0.0), 30.0)
    qy = jnp.minimum(jnp.maximum(jnp.abs(y) * 4.0 - 3.0, 0.0), 30.0)
    s = x * x + y * y
    si = lax.bitcast_convert_type(s, jnp.int32)
    r = lax.bitcast_convert_type(_MAGIC - (si >> 1), jnp.float32)
    r = r * (1.5 - (0.5 * s) * r * r)
    rt = s * r
    j = jnp.minimum(jnp.maximum(rt * 8.0 - 1.0, 0.0), 20.0)
    return 0.1 * (qx * qx + qy * qy + j * j)


_mesh = plsc.VectorSubcoreMesh(core_axis_name="c", subcore_axis_name="s")


@functools.partial(
    pl.kernel,
    mesh=_mesh,
    out_type=jax.ShapeDtypeStruct((8, 128, 8, 128), jnp.float32),
    scratch_types=[
        pltpu.VMEM((2, 8, 8, 2, 128), jnp.float32),
        pltpu.VMEM((2, 8, 8, 128), jnp.float32),
        pltpu.SemaphoreType.DMA,
        pltpu.SemaphoreType.DMA,
        pltpu.SemaphoreType.DMA,
        pltpu.SemaphoreType.DMA,
    ],
)
def _sc_cost(x4, o4, in_buf, out_buf, si0, si1, so0, so1):
    wid = lax.axis_index("s") * _NC + lax.axis_index("c")
    rtile = wid // 4
    quarter = wid % 4
    in_sems = (si0, si1)
    out_sems = (so0, so1)

    def start_in(g):
        ct = quarter * 32 + g * 8
        return pltpu.async_copy(
            x4.at[pl.ds(rtile * 8, 8), pl.ds(ct, 8)],
            in_buf.at[g % 2], in_sems[g % 2])

    out_dmas = [None, None]
    in_dmas = [start_in(0), None]
    for g in range(4):
        if g < 3:
            in_dmas[(g + 1) % 2] = start_in(g + 1)
        in_dmas[g % 2].wait()
        if out_dmas[g % 2] is not None:
            out_dmas[g % 2].wait()

        @plsc.parallel_loop(0, 64, unroll=2)
        def body(k):
            i = k // 8
            j = k % 8
            for l in range(8):
                x = in_buf[g % 2, i, j, 0, pl.ds(l * 16, 16)]
                y = in_buf[g % 2, i, j, 1, pl.ds(l * 16, 16)]
                out_buf[g % 2, j, i, pl.ds(l * 16, 16)] = _cost16(x, y)

        ct = quarter * 32 + g * 8
        out_dmas[g % 2] = pltpu.async_copy(
            out_buf.at[g % 2], o4.at[rtile, pl.ds(ct, 8)], out_sems[g % 2])
    out_dmas[0].wait()
    out_dmas[1].wait()


def kernel(trajs):
    b, n, _ = trajs.shape
    x4 = trajs.reshape(b, n // 128, 128, 2).transpose(0, 1, 3, 2)
    o4 = _sc_cost(x4)
    return o4.transpose(0, 2, 1, 3).reshape(b, n)


# speedup vs baseline: 2.8386x; 1.5223x over previous
import functools

import jax
import jax.numpy as jnp
from jax import lax
from jax.experimental import pallas as pl
from jax.experimental.pallas import tpu as pltpu
from jax.experimental.pallas import tpu_sc as plsc

_NC = 2
_NS = 16
_MAGIC = 0x5F3759DF
_BN = 8192
_TC_B = 56
_SC_RT = _TC_B // 8


def _tc_body(a_ref, o_ref):
    x = a_ref[:, 0, :]
    y = a_ref[:, 1, :]
    qx = jnp.clip(jnp.abs(4.0 * x) - 3.0, 0.0, 30.0)
    qy = jnp.clip(jnp.abs(4.0 * y) - 3.0, 0.0, 30.0)
    s = x * x + y * y
    jerk = jnp.clip(8.0 * jnp.sqrt(s) - 1.0, 0.0, 20.0)
    o_ref[...] = 0.1 * (qx * qx + qy * qy + jerk * jerk)


def _cost16(x, y):
    qx = jnp.minimum(jnp.maximum(jnp.abs(x) * 4.0 - 3.0, 0.0), 30.0)
    qy = jnp.minimum(jnp.maximum(jnp.abs(y) * 4.0 - 3.0, 0.0), 30.0)
    s = x * x + y * y
    si = lax.bitcast_convert_type(s, jnp.int32)
    r = lax.bitcast_convert_type(_MAGIC - (si >> 1), jnp.float32)
    r = r * (1.5 - (0.5 * s) * r * r)
    rt = s * r
    j = jnp.minimum(jnp.maximum(rt * 8.0 - 1.0, 0.0), 20.0)
    return 0.1 * (qx * qx + qy * qy + j * j)


_mesh = plsc.VectorSubcoreMesh(core_axis_name="c", subcore_axis_name="s")


@functools.partial(
    pl.kernel,
    mesh=_mesh,
    out_type=jax.ShapeDtypeStruct((1, 128, 8, 128), jnp.float32),
    scratch_types=[
        pltpu.VMEM((8, 4, 2, 128), jnp.float32),
        pltpu.VMEM((4, 8, 128), jnp.float32),
        pltpu.SemaphoreType.DMA,
    ],
)
def _sc_cost(x4, o4, in_buf, out_buf, sem):
    wid = lax.axis_index("s") * _NC + lax.axis_index("c")
    ct = wid * 4
    pltpu.async_copy(
        x4.at[pl.ds(_SC_RT * 8, 8), pl.ds(ct, 4)], in_buf, sem).wait()

    @plsc.parallel_loop(0, 32, unroll=4)
    def body(k):
        i = k // 4
        j = k % 4
        for l in range(8):
            x = in_buf[i, j, 0, pl.ds(l * 16, 16)]
            y = in_buf[i, j, 1, pl.ds(l * 16, 16)]
            out_buf[j, i, pl.ds(l * 16, 16)] = _cost16(x, y)

    pltpu.async_copy(out_buf, o4.at[0, pl.ds(ct, 4)], sem).wait()


def kernel(trajs):
    b, n, _ = trajs.shape
    x4 = trajs.reshape(b, n // 128, 128, 2).transpose(0, 1, 3, 2)
    o4 = _sc_cost(x4)
    sc_out = o4.transpose(0, 2, 1, 3).reshape(8, n)

    planes = jnp.transpose(trajs, (0, 2, 1))
    tc_out = pl.pallas_call(
        _tc_body,
        grid=(n // _BN,),
        in_specs=[pl.BlockSpec((_TC_B, 2, _BN), lambda i: (0, 0, i))],
        out_specs=pl.BlockSpec((_TC_B, _BN), lambda i: (0, i)),
        out_shape=jax.ShapeDtypeStruct((_TC_B, n), jnp.float32),
    )(planes)
    return jnp.concatenate([tc_out, sc_out], axis=0)


# speedup vs baseline: 11.1261x; 3.9195x over previous
import jax
import jax.numpy as jnp
from jax import lax
from jax.experimental import pallas as pl

_BN = 8192


def _body(a_ref, o_ref):
    x = a_ref[:, 0, :]
    y = a_ref[:, 1, :]
    cx = jnp.clip(jnp.abs(x) - 0.75, 0.0, 7.5)
    cy = jnp.clip(jnp.abs(y) - 0.75, 0.0, 7.5)
    s = x * x + y * y
    rt = s * lax.rsqrt(jnp.maximum(s, 1e-35))
    cj = jnp.clip(rt - 0.125, 0.0, 2.5)
    o_ref[...] = 1.6 * (cx * cx + cy * cy) + 6.4 * (cj * cj)


def kernel(trajs):
    b, n, _ = trajs.shape
    planes = jnp.transpose(trajs, (0, 2, 1))
    out = pl.pallas_call(
        _body,
        grid=(n // _BN,),
        in_specs=[pl.BlockSpec((b, 2, _BN), lambda i: (0, 0, i))],
        out_specs=pl.BlockSpec((b, _BN), lambda i: (0, i)),
        out_shape=jax.ShapeDtypeStruct((b, n), jnp.float32),
    )(planes)
    return out
